# initial kernel scaffold (unmeasured)
import jax
import jax.numpy as jnp
from jax import lax
from jax.experimental import pallas as pl
from jax.experimental.pallas import tpu as pltpu

N_DEV = 4
TAPS = 4
HALO = TAPS - 1


def kernel(x, k):
    B, S, C = x.shape
    BLK = 512
    n_blk = S // BLK

    def body(x_ref, x_hbm, k_ref, out_ref, tail_ref, send_sem, recv_sem,
             exit_sem):
        j = pl.program_id(0)
        my = lax.axis_index("i")
        n_partners = (
            jnp.where(my > 0, 1, 0) + jnp.where(my < N_DEV - 1, 1, 0)
        )

        @pl.when(j == 0)
        def _():
            barrier = pltpu.get_barrier_semaphore()

            @pl.when(my > 0)
            def _():
                pl.semaphore_signal(
                    barrier, inc=1, device_id=(my - 1,),
                    device_id_type=pl.DeviceIdType.MESH,
                )

            @pl.when(my < N_DEV - 1)
            def _():
                pl.semaphore_signal(
                    barrier, inc=1, device_id=(my + 1,),
                    device_id_type=pl.DeviceIdType.MESH,
                )

            pl.semaphore_wait(barrier, n_partners)

            rdma = pltpu.make_async_remote_copy(
                src_ref=x_hbm.at[:, pl.ds(S - HALO, HALO), :],
                dst_ref=tail_ref,
                send_sem=send_sem,
                recv_sem=recv_sem,
                device_id=((my + 1) % N_DEV,),
                device_id_type=pl.DeviceIdType.MESH,
            )

            @pl.when(my < N_DEV - 1)
            def _():
                rdma.start()

            @pl.when(my > 0)
            def _():
                rdma.wait_recv()

            @pl.when(my == 0)
            def _():
                tail_ref[...] = jnp.zeros_like(tail_ref)

            @pl.when(my < N_DEV - 1)
            def _():
                rdma.wait_send()

        prev = tail_ref[...]
        xb = x_ref[...]
        ext = jnp.concatenate([prev, xb], axis=1)
        kv = k_ref[...]
        acc = xb * kv[TAPS - 1].reshape(1, 1, C)
        for t in range(TAPS - 1):
            acc += ext[:, t:t + BLK, :] * kv[t].reshape(1, 1, C)
        out_ref[...] = (acc * jax.nn.sigmoid(acc)).astype(out_ref.dtype)

        tail_ref[...] = xb[:, BLK - HALO:, :]

        @pl.when(j == n_blk - 1)
        def _():
            @pl.when(my > 0)
            def _():
                pl.semaphore_signal(
                    exit_sem, inc=1, device_id=(my - 1,),
                    device_id_type=pl.DeviceIdType.MESH,
                )

            @pl.when(my < N_DEV - 1)
            def _():
                pl.semaphore_signal(
                    exit_sem, inc=1, device_id=(my + 1,),
                    device_id_type=pl.DeviceIdType.MESH,
                )

            pl.semaphore_wait(exit_sem, n_partners)

    return pl.pallas_call(
        body,
        grid=(n_blk,),
        in_specs=[
            pl.BlockSpec((B, BLK, C), lambda j: (0, j, 0)),
            pl.BlockSpec(memory_space=pltpu.ANY),
            pl.BlockSpec((TAPS, C), lambda j: (0, 0)),
        ],
        out_specs=pl.BlockSpec((B, BLK, C), lambda j: (0, j, 0)),
        out_shape=jax.ShapeDtypeStruct((B, S, C), jnp.bfloat16),
        scratch_shapes=[
            pltpu.VMEM((B, HALO, C), jnp.float32),
            pltpu.SemaphoreType.DMA,
            pltpu.SemaphoreType.DMA,
            pltpu.SemaphoreType.REGULAR,
        ],
        compiler_params=pltpu.CompilerParams(
            collective_id=0,
            dimension_semantics=("arbitrary",),
        ),
    )(x, x, k)


# baseline (device time: 39527 ns/iter reference)
import jax
import jax.numpy as jnp
from jax import lax
from jax.experimental import pallas as pl
from jax.experimental.pallas import tpu as pltpu

N_DEV = 4
TAPS = 4
HALO = TAPS - 1


def kernel(x, k):
    B, S, C = x.shape
    BLK = 512
    n_blk = S // BLK

    def body(x_ref, x_hbm, k_ref, out_ref, tail_ref, send_sem, recv_sem,
             exit_sem):
        j = pl.program_id(0)
        my = lax.axis_index("i")
        n_partners = (
            jnp.where(my > 0, 1, 0) + jnp.where(my < N_DEV - 1, 1, 0)
        )

        @pl.when(j == 0)
        def _():
            barrier = pltpu.get_barrier_semaphore()

            @pl.when(my > 0)
            def _():
                pl.semaphore_signal(
                    barrier, inc=1, device_id=(my - 1,),
                    device_id_type=pl.DeviceIdType.MESH,
                )

            @pl.when(my < N_DEV - 1)
            def _():
                pl.semaphore_signal(
                    barrier, inc=1, device_id=(my + 1,),
                    device_id_type=pl.DeviceIdType.MESH,
                )

            pl.semaphore_wait(barrier, n_partners)

            rdma = pltpu.make_async_remote_copy(
                src_ref=x_hbm.at[:, pl.ds(S - HALO, HALO), :],
                dst_ref=tail_ref,
                send_sem=send_sem,
                recv_sem=recv_sem,
                device_id=((my + 1) % N_DEV,),
                device_id_type=pl.DeviceIdType.MESH,
            )

            @pl.when(my < N_DEV - 1)
            def _():
                rdma.start()

            @pl.when(my > 0)
            def _():
                rdma.wait_recv()

            @pl.when(my == 0)
            def _():
                tail_ref[...] = jnp.zeros_like(tail_ref)

            @pl.when(my < N_DEV - 1)
            def _():
                rdma.wait_send()

        prev = tail_ref[...]
        xb = x_ref[...]
        ext = jnp.concatenate([prev, xb], axis=1)
        kv = k_ref[...]
        acc = xb * kv[TAPS - 1].reshape(1, 1, C)
        for t in range(TAPS - 1):
            acc += ext[:, t:t + BLK, :] * kv[t].reshape(1, 1, C)
        out_ref[...] = (acc * jax.nn.sigmoid(acc)).astype(out_ref.dtype)

        tail_ref[...] = xb[:, BLK - HALO:, :]

        @pl.when(j == n_blk - 1)
        def _():
            @pl.when(my > 0)
            def _():
                pl.semaphore_signal(
                    exit_sem, inc=1, device_id=(my - 1,),
                    device_id_type=pl.DeviceIdType.MESH,
                )

            @pl.when(my < N_DEV - 1)
            def _():
                pl.semaphore_signal(
                    exit_sem, inc=1, device_id=(my + 1,),
                    device_id_type=pl.DeviceIdType.MESH,
                )

            pl.semaphore_wait(exit_sem, n_partners)

    return pl.pallas_call(
        body,
        grid=(n_blk,),
        in_specs=[
            pl.BlockSpec((B, BLK, C), lambda j: (0, j, 0)),
            pl.BlockSpec(memory_space=pl.ANY),
            pl.BlockSpec((TAPS, C), lambda j: (0, 0)),
        ],
        out_specs=pl.BlockSpec((B, BLK, C), lambda j: (0, j, 0)),
        out_shape=jax.ShapeDtypeStruct((B, S, C), jnp.bfloat16),
        scratch_shapes=[
            pltpu.VMEM((B, HALO, C), jnp.float32),
            pltpu.SemaphoreType.DMA,
            pltpu.SemaphoreType.DMA,
            pltpu.SemaphoreType.REGULAR,
        ],
        compiler_params=pltpu.CompilerParams(
            collective_id=0,
            dimension_semantics=("arbitrary",),
        ),
    )(x, x, k)


# device time: 30449 ns/iter; 1.2981x vs baseline; 1.2981x over previous
import jax
import jax.numpy as jnp
from jax import lax
from jax.experimental import pallas as pl
from jax.experimental.pallas import tpu as pltpu

N_DEV = 4
TAPS = 4
HALO = TAPS - 1


def kernel(x, k):
    B, S, C = x.shape
    BLK = 512
    n_blk = S // BLK

    def body(x_ref, x_hbm, k_ref, out_ref, tail_ref, send_sem, recv_sem,
             exit_sem):
        j = pl.program_id(0)
        my = lax.axis_index("i")
        n_partners = (
            jnp.where(my > 0, 1, 0) + jnp.where(my < N_DEV - 1, 1, 0)
        )

        @pl.when(j == 0)
        def _():
            barrier = pltpu.get_barrier_semaphore()

            @pl.when(my > 0)
            def _():
                pl.semaphore_signal(
                    barrier, inc=1, device_id=(my - 1,),
                    device_id_type=pl.DeviceIdType.MESH,
                )

            @pl.when(my < N_DEV - 1)
            def _():
                pl.semaphore_signal(
                    barrier, inc=1, device_id=(my + 1,),
                    device_id_type=pl.DeviceIdType.MESH,
                )

            pl.semaphore_wait(barrier, n_partners)

            rdma = pltpu.make_async_remote_copy(
                src_ref=x_hbm.at[:, pl.ds(S - HALO, HALO), :],
                dst_ref=tail_ref,
                send_sem=send_sem,
                recv_sem=recv_sem,
                device_id=((my + 1) % N_DEV,),
                device_id_type=pl.DeviceIdType.MESH,
            )

            @pl.when(my < N_DEV - 1)
            def _():
                rdma.start()

            @pl.when(my > 0)
            def _():
                rdma.wait_recv()

            @pl.when(my == 0)
            def _():
                tail_ref[...] = jnp.zeros_like(tail_ref)

            @pl.when(my < N_DEV - 1)
            def _():
                rdma.wait_send()

        prev = tail_ref[...]
        xb = x_ref[...]
        xb16 = xb.astype(jnp.bfloat16)
        ext = jnp.concatenate(
            [prev.astype(jnp.bfloat16), xb16], axis=1
        )
        kv = k_ref[...].astype(jnp.bfloat16)
        acc = xb16 * kv[TAPS - 1].reshape(1, 1, C)
        for t in range(TAPS - 1):
            acc += ext[:, t:t + BLK, :] * kv[t].reshape(1, 1, C)
        out_ref[...] = acc * jax.nn.sigmoid(acc)

        tail_ref[...] = xb[:, BLK - HALO:, :]

        @pl.when(j == n_blk - 1)
        def _():
            @pl.when(my > 0)
            def _():
                pl.semaphore_signal(
                    exit_sem, inc=1, device_id=(my - 1,),
                    device_id_type=pl.DeviceIdType.MESH,
                )

            @pl.when(my < N_DEV - 1)
            def _():
                pl.semaphore_signal(
                    exit_sem, inc=1, device_id=(my + 1,),
                    device_id_type=pl.DeviceIdType.MESH,
                )

            pl.semaphore_wait(exit_sem, n_partners)

    return pl.pallas_call(
        body,
        grid=(n_blk,),
        in_specs=[
            pl.BlockSpec((B, BLK, C), lambda j: (0, j, 0)),
            pl.BlockSpec(memory_space=pl.ANY),
            pl.BlockSpec((TAPS, C), lambda j: (0, 0)),
        ],
        out_specs=pl.BlockSpec((B, BLK, C), lambda j: (0, j, 0)),
        out_shape=jax.ShapeDtypeStruct((B, S, C), jnp.bfloat16),
        scratch_shapes=[
            pltpu.VMEM((B, HALO, C), jnp.float32),
            pltpu.SemaphoreType.DMA,
            pltpu.SemaphoreType.DMA,
            pltpu.SemaphoreType.REGULAR,
        ],
        compiler_params=pltpu.CompilerParams(
            collective_id=0,
            dimension_semantics=("arbitrary",),
        ),
    )(x, x, k)


# device time: 30108 ns/iter; 1.3128x vs baseline; 1.0113x over previous
import jax
import jax.numpy as jnp
from jax import lax
from jax.experimental import pallas as pl
from jax.experimental.pallas import tpu as pltpu

N_DEV = 4
TAPS = 4
HALO = TAPS - 1


def kernel(x, k):
    B, S, C = x.shape
    BLK = 512
    n_blk = S // BLK

    def body(x_ref, x_hbm, k_ref, out_ref, tail_ref, halo_ref,
             send_sem, recv_sem, local_sem):
        j = pl.program_id(0)
        my = lax.axis_index("i")

        def halo_rdma():
            return pltpu.make_async_remote_copy(
                src_ref=x_hbm.at[:, pl.ds(S - HALO, HALO), :],
                dst_ref=halo_ref,
                send_sem=send_sem,
                recv_sem=recv_sem,
                device_id=((my + 1) % N_DEV,),
                device_id_type=pl.DeviceIdType.MESH,
            )

        @pl.when(j == 0)
        def _():
            barrier = pltpu.get_barrier_semaphore()

            @pl.when(my > 0)
            def _():
                pl.semaphore_signal(
                    barrier, inc=1, device_id=(my - 1,),
                    device_id_type=pl.DeviceIdType.MESH,
                )

            @pl.when(my < N_DEV - 1)
            def _():
                pl.semaphore_signal(
                    barrier, inc=1, device_id=(my + 1,),
                    device_id_type=pl.DeviceIdType.MESH,
                )

            n_partners = (
                jnp.where(my > 0, 1, 0) + jnp.where(my < N_DEV - 1, 1, 0)
            )
            pl.semaphore_wait(barrier, n_partners)

            @pl.when(my < N_DEV - 1)
            def _():
                halo_rdma().start()

            @pl.when(my == 0)
            def _():
                halo_ref[...] = jnp.zeros_like(halo_ref)

            cp = pltpu.make_async_copy(
                x_hbm.at[:, pl.ds(BLK - HALO, HALO), :], tail_ref,
                local_sem,
            )
            cp.start()
            cp.wait()

        @pl.when(j == n_blk - 1)
        def _():
            rdma = halo_rdma()

            @pl.when(my > 0)
            def _():
                rdma.wait_recv()

            @pl.when(my < N_DEV - 1)
            def _():
                rdma.wait_send()

        is_last = j == n_blk - 1
        prev = jnp.where(is_last, halo_ref[...], tail_ref[...])
        xb = x_ref[...]
        xb16 = xb.astype(jnp.bfloat16)
        ext = jnp.concatenate(
            [prev.astype(jnp.bfloat16), xb16], axis=1
        )
        kv = k_ref[...].astype(jnp.bfloat16)
        acc = xb16 * kv[TAPS - 1].reshape(1, 1, C)
        for t in range(TAPS - 1):
            acc += ext[:, t:t + BLK, :] * kv[t].reshape(1, 1, C)
        out_ref[...] = acc * jax.nn.sigmoid(acc)

        tail_ref[...] = xb[:, BLK - HALO:, :]

    return pl.pallas_call(
        body,
        grid=(n_blk,),
        in_specs=[
            pl.BlockSpec((B, BLK, C), lambda j: (0, (j + 1) % n_blk, 0)),
            pl.BlockSpec(memory_space=pl.ANY),
            pl.BlockSpec((TAPS, C), lambda j: (0, 0)),
        ],
        out_specs=pl.BlockSpec((B, BLK, C), lambda j: (0, (j + 1) % n_blk, 0)),
        out_shape=jax.ShapeDtypeStruct((B, S, C), jnp.bfloat16),
        scratch_shapes=[
            pltpu.VMEM((B, HALO, C), jnp.float32),
            pltpu.VMEM((B, HALO, C), jnp.float32),
            pltpu.SemaphoreType.DMA,
            pltpu.SemaphoreType.DMA,
            pltpu.SemaphoreType.DMA,
        ],
        compiler_params=pltpu.CompilerParams(
            collective_id=0,
            dimension_semantics=("arbitrary",),
        ),
    )(x, x, k)


# device time: 27679 ns/iter; 1.4281x vs baseline; 1.0878x over previous
import jax
import jax.numpy as jnp
from jax import lax
from jax.experimental import pallas as pl
from jax.experimental.pallas import tpu as pltpu

N_DEV = 4
TAPS = 4
HALO = TAPS - 1


def kernel(x, k):
    B, S, C = x.shape
    BLK = 512
    n_blk = S // BLK

    def body(x_ref, x_hbm, k_ref, out_ref, tail_ref, halo_ref,
             send_sem, recv_sem, local_sem):
        j = pl.program_id(0)
        my = lax.axis_index("i")

        def halo_rdma():
            return pltpu.make_async_remote_copy(
                src_ref=x_hbm.at[:, pl.ds(S - HALO, HALO), :],
                dst_ref=halo_ref,
                send_sem=send_sem,
                recv_sem=recv_sem,
                device_id=((my + 1) % N_DEV,),
                device_id_type=pl.DeviceIdType.MESH,
            )

        @pl.when(j == 0)
        def _():
            barrier = pltpu.get_barrier_semaphore()

            @pl.when(my > 0)
            def _():
                pl.semaphore_signal(
                    barrier, inc=1, device_id=(my - 1,),
                    device_id_type=pl.DeviceIdType.MESH,
                )

            @pl.when(my < N_DEV - 1)
            def _():
                pl.semaphore_signal(
                    barrier, inc=1, device_id=(my + 1,),
                    device_id_type=pl.DeviceIdType.MESH,
                )

            @pl.when(my == 0)
            def _():
                halo_ref[...] = jnp.zeros_like(halo_ref)

            cp = pltpu.make_async_copy(
                x_hbm.at[:, pl.ds(BLK - HALO, HALO), :], tail_ref,
                local_sem,
            )
            cp.start()
            cp.wait()

        @pl.when(j == 2)
        def _():
            barrier = pltpu.get_barrier_semaphore()
            n_partners = (
                jnp.where(my > 0, 1, 0) + jnp.where(my < N_DEV - 1, 1, 0)
            )
            pl.semaphore_wait(barrier, n_partners)

            @pl.when(my < N_DEV - 1)
            def _():
                halo_rdma().start()

        @pl.when(j == n_blk - 1)
        def _():
            rdma = halo_rdma()

            @pl.when(my > 0)
            def _():
                rdma.wait_recv()

            @pl.when(my < N_DEV - 1)
            def _():
                rdma.wait_send()

        is_last = j == n_blk - 1
        prev = jnp.where(is_last, halo_ref[...], tail_ref[...])
        xb = x_ref[...]
        xb16 = xb.astype(jnp.bfloat16)
        ext = jnp.concatenate(
            [prev.astype(jnp.bfloat16), xb16], axis=1
        )
        kv = k_ref[...].astype(jnp.bfloat16)
        acc = xb16 * kv[TAPS - 1].reshape(1, 1, C)
        for t in range(TAPS - 1):
            acc += ext[:, t:t + BLK, :] * kv[t].reshape(1, 1, C)
        out_ref[...] = acc * jax.nn.sigmoid(acc)

        tail_ref[...] = xb[:, BLK - HALO:, :]

    return pl.pallas_call(
        body,
        grid=(n_blk,),
        in_specs=[
            pl.BlockSpec((B, BLK, C), lambda j: (0, (j + 1) % n_blk, 0)),
            pl.BlockSpec(memory_space=pl.ANY),
            pl.BlockSpec((TAPS, C), lambda j: (0, 0)),
        ],
        out_specs=pl.BlockSpec((B, BLK, C), lambda j: (0, (j + 1) % n_blk, 0)),
        out_shape=jax.ShapeDtypeStruct((B, S, C), jnp.bfloat16),
        scratch_shapes=[
            pltpu.VMEM((B, HALO, C), jnp.float32),
            pltpu.VMEM((B, HALO, C), jnp.float32),
            pltpu.SemaphoreType.DMA,
            pltpu.SemaphoreType.DMA,
            pltpu.SemaphoreType.DMA,
        ],
        compiler_params=pltpu.CompilerParams(
            collective_id=0,
            dimension_semantics=("arbitrary",),
        ),
    )(x, x, k)
